# baseline (speedup 1.0000x reference)
import functools

import jax
import jax.numpy as jnp
from jax import lax
from jax.experimental import pallas as pl
from jax.experimental.pallas import tpu as pltpu
from jax.experimental.pallas import tpu_sc as plsc

OUT_H, OUT_W = 7, 7
NBIN = OUT_H * OUT_W
SPATIAL_SCALE = 1.0 / 16.0
WIN = 8
NW = 32
HDR = 16
ROWS0 = HDR
CORN0 = HDR + WIN * WIN
META_W = CORN0 + 16 * NBIN


def _sc_body(meta_hbm, feat_hbm, out_hbm, m0, m1, reg0, reg1, o0, o1,
             gs0, gs1, os0, os1):
    C = feat_hbm.shape[1]
    NCH = C // 16
    N = meta_hbm.shape[0]
    ms, regs, outs = [m0, m1], [reg0, reg1], [o0, o1]
    gsems, osems = [gs0, gs1], [os0, os1]
    wid = lax.axis_index("s") * 2 + lax.axis_index("c")
    count = (N - 1 - wid) // NW + 1

    def issue(t, b):
        pltpu.sync_copy(meta_hbm.at[t * NW + wid], ms[b])
        k = ms[b][pl.ds(0, 16)][0]
        for cls in range(4):
            L = 16 * (cls + 1)

            @pl.when(k == cls)
            def _issue_cls(L=L, b=b):
                pltpu.async_copy(
                    feat_hbm.at[ms[b].at[pl.ds(ROWS0, L)]],
                    regs[b].at[pl.ds(0, L)],
                    gsems[b],
                )

    def gather_wait(b):
        k = ms[b][pl.ds(0, 16)][0]
        for cls in range(4):
            L = 16 * (cls + 1)

            @pl.when(k == cls)
            def _wait_cls(L=L, b=b):
                pltpu.make_async_copy(
                    feat_hbm.at[ms[b].at[pl.ds(ROWS0, L)]],
                    regs[b].at[pl.ds(0, L)],
                    gsems[b],
                ).wait()

    def out_wait(b):
        pltpu.make_async_copy(outs[b], out_hbm.at[0], osems[b]).wait()

    @pl.when(count > 0)
    def _prologue():
        issue(0, 0)

    def pair_body(g, carry):
        for b in range(2):
            t = 2 * g + b
            nxt = 1 - b

            @pl.when(t + 1 < count)
            def _prefetch():
                issue(t + 1, nxt)

            @pl.when(t < count)
            def _process():
                gather_wait(b)

                @pl.when(t >= 2)
                def _drain():
                    out_wait(b)

                meta_v, reg_v, out_v = ms[b], regs[b], outs[b]

                def bin_body(ij, carry2):
                    cv = meta_v[pl.ds(CORN0 + ij * 16, 16)]
                    c0, c1, c2, c3 = cv[0], cv[1], cv[2], cv[3]
                    for c in range(NCH):
                        sl = pl.ds(c * 16, 16)
                        v = jnp.maximum(
                            jnp.maximum(reg_v[c0, sl], reg_v[c1, sl]),
                            jnp.maximum(reg_v[c2, sl], reg_v[c3, sl]),
                        )
                        out_v[ij, sl] = v
                    return carry2

                lax.fori_loop(0, NBIN, bin_body, 0)
                pltpu.async_copy(out_v, out_hbm.at[t * NW + wid], osems[b])

        return carry

    lax.fori_loop(0, (count + 1) // 2, pair_body, 0)

    @pl.when(count >= 1)
    def _drain0():
        out_wait(0)

    @pl.when(count >= 2)
    def _drain1():
        out_wait(1)


@jax.jit
def _roi_pool_sc(feat_rows, meta):
    R, C = feat_rows.shape
    N = meta.shape[0]
    mesh = plsc.VectorSubcoreMesh(core_axis_name="c", subcore_axis_name="s")
    f = functools.partial(
        pl.kernel,
        mesh=mesh,
        compiler_params=pltpu.CompilerParams(
            needs_layout_passes=False, use_tc_tiling_on_sc=False
        ),
        out_type=jax.ShapeDtypeStruct((N, NBIN, C), jnp.float32),
        scratch_types=[
            pltpu.VMEM((META_W,), jnp.int32),
            pltpu.VMEM((META_W,), jnp.int32),
            pltpu.VMEM((WIN * WIN, C), jnp.float32),
            pltpu.VMEM((WIN * WIN, C), jnp.float32),
            pltpu.VMEM((NBIN, C), jnp.float32),
            pltpu.VMEM((NBIN, C), jnp.float32),
            pltpu.SemaphoreType.DMA,
            pltpu.SemaphoreType.DMA,
            pltpu.SemaphoreType.DMA,
            pltpu.SemaphoreType.DMA,
        ],
    )(_sc_body)
    return f(meta, feat_rows)


def kernel(features, rois, roi_indices):
    B, C, H, W = features.shape
    N = rois.shape[0]
    rois_i = (rois * SPATIAL_SCALE).astype(jnp.int32)
    img = roi_indices.astype(jnp.int32)
    hx, wy = rois_i[:, 0], rois_i[:, 1]
    lh = jnp.clip(rois_i[:, 2] - hx, 1, WIN)
    lw = jnp.clip(rois_i[:, 3] - wy, 1, WIN)
    ncells = lh * lw
    cls = (ncells + 15) // 16 - 1

    i8 = jnp.arange(WIN, dtype=jnp.int32)
    ii = i8[None, :, None]
    jj = i8[None, None, :]
    pos = ii * lw[:, None, None] + jj
    valid = (ii < lh[:, None, None]) & (jj < lw[:, None, None])
    rowid = (img * (H * W) + hx * W + wy)[:, None, None] + ii * W + jj
    rowid = jnp.clip(rowid, 0, B * H * W - 1)
    pos = jnp.where(valid, pos, WIN * WIN)
    base = jnp.broadcast_to(
        (img * (H * W) + hx * W + wy)[:, None], (N, WIN * WIN + 1)
    )
    nidx = jnp.broadcast_to(jnp.arange(N, dtype=jnp.int32)[:, None, None],
                            (N, WIN, WIN))
    idx_rows = base.at[nidx, pos].set(rowid)[:, : WIN * WIN]

    def bounds(l, n_out):
        i = jnp.arange(n_out, dtype=jnp.int32)[None, :]
        r0 = (i * l[:, None]) // n_out
        r1m = -(((-(i + 1)) * l[:, None]) // n_out) - 1
        r1m = jnp.clip(jnp.maximum(r1m, r0), 0, None)
        return r0, jnp.minimum(r1m, l[:, None] - 1)

    x0, x1 = bounds(lh, OUT_H)
    y0, y1 = bounds(lw, OUT_W)
    pw = lw[:, None, None]
    corners = jnp.stack(
        [
            x0[:, :, None] * pw + y0[:, None, :],
            x0[:, :, None] * pw + y1[:, None, :],
            x1[:, :, None] * pw + y0[:, None, :],
            x1[:, :, None] * pw + y1[:, None, :],
        ],
        axis=3,
    ).reshape(N, NBIN, 4)
    cells = jnp.zeros((N, NBIN, 16), jnp.int32).at[:, :, :4].set(corners)
    hdr = jnp.zeros((N, HDR), jnp.int32).at[:, 0].set(cls)
    meta = jnp.concatenate([hdr, idx_rows, cells.reshape(N, 16 * NBIN)], axis=1)

    feat_rows = features.transpose(0, 2, 3, 1).reshape(B * H * W, C)
    out = _roi_pool_sc(feat_rows, meta)
    return out.transpose(0, 2, 1).reshape(N, C, OUT_H, OUT_W)


# speedup vs baseline: 1.0203x; 1.0203x over previous
import functools

import jax
import jax.numpy as jnp
from jax import lax
from jax.experimental import pallas as pl
from jax.experimental.pallas import tpu as pltpu
from jax.experimental.pallas import tpu_sc as plsc

OUT_H, OUT_W = 7, 7
NBIN = OUT_H * OUT_W
SPATIAL_SCALE = 1.0 / 16.0
WIN = 8
NW = 32
HDR = 16
ROWS0 = HDR
CORN0 = HDR + WIN * WIN
META_W = CORN0 + 16 * (NBIN + 1)


def _sc_body(meta_hbm, feat_hbm, out_hbm, m0, m1, reg0, reg1, o0, o1,
             gs0, gs1, os0, os1):
    C = feat_hbm.shape[1]
    NCH = C // 16
    N = meta_hbm.shape[0]
    ms, regs, outs = [m0, m1], [reg0, reg1], [o0, o1]
    gsems, osems = [gs0, gs1], [os0, os1]
    wid = lax.axis_index("s") * 2 + lax.axis_index("c")
    count = (N - 1 - wid) // NW + 1

    def issue(t, b):
        pltpu.sync_copy(meta_hbm.at[t * NW + wid], ms[b])
        k = ms[b][pl.ds(0, 16)][0]
        for cls in range(4):
            L = 16 * (cls + 1)

            @pl.when(k == cls)
            def _issue_cls(L=L, b=b):
                pltpu.async_copy(
                    feat_hbm.at[ms[b].at[pl.ds(ROWS0, L)]],
                    regs[b].at[pl.ds(0, L)],
                    gsems[b],
                )

    def gather_wait(b):
        k = ms[b][pl.ds(0, 16)][0]
        for cls in range(4):
            L = 16 * (cls + 1)

            @pl.when(k == cls)
            def _wait_cls(L=L, b=b):
                pltpu.make_async_copy(
                    feat_hbm.at[ms[b].at[pl.ds(ROWS0, L)]],
                    regs[b].at[pl.ds(0, L)],
                    gsems[b],
                ).wait()

    def out_wait(b):
        pltpu.make_async_copy(outs[b], out_hbm.at[0], osems[b]).wait()

    @pl.when(count > 0)
    def _prologue():
        issue(0, 0)

    def pair_body(g, carry):
        for b in range(2):
            t = 2 * g + b
            nxt = 1 - b

            @pl.when(t + 1 < count)
            def _prefetch():
                issue(t + 1, nxt)

            @pl.when(t < count)
            def _process():
                gather_wait(b)

                @pl.when(t >= 2)
                def _drain():
                    out_wait(b)

                meta_v, reg_v, out_v = ms[b], regs[b], outs[b]

                def corners_of(ij):
                    cv = meta_v[pl.ds(CORN0 + ij * 16, 16)]
                    return cv[0], cv[1], cv[2], cv[3]

                def bin_body(ij, carry2):
                    c0, c1, c2, c3 = carry2
                    nxt_c = corners_of(ij + 1)
                    for c in range(NCH):
                        sl = pl.ds(c * 16, 16)
                        v = jnp.maximum(
                            jnp.maximum(reg_v[c0, sl], reg_v[c1, sl]),
                            jnp.maximum(reg_v[c2, sl], reg_v[c3, sl]),
                        )
                        out_v[ij, sl] = v
                    return nxt_c

                lax.fori_loop(0, NBIN, bin_body, corners_of(0))
                pltpu.async_copy(out_v, out_hbm.at[t * NW + wid], osems[b])

        return carry

    lax.fori_loop(0, (count + 1) // 2, pair_body, 0)

    @pl.when(count >= 1)
    def _drain0():
        out_wait(0)

    @pl.when(count >= 2)
    def _drain1():
        out_wait(1)


@jax.jit
def _roi_pool_sc(feat_rows, meta):
    R, C = feat_rows.shape
    N = meta.shape[0]
    mesh = plsc.VectorSubcoreMesh(core_axis_name="c", subcore_axis_name="s")
    f = functools.partial(
        pl.kernel,
        mesh=mesh,
        compiler_params=pltpu.CompilerParams(
            needs_layout_passes=False, use_tc_tiling_on_sc=False
        ),
        out_type=jax.ShapeDtypeStruct((N, NBIN, C), jnp.float32),
        scratch_types=[
            pltpu.VMEM((META_W,), jnp.int32),
            pltpu.VMEM((META_W,), jnp.int32),
            pltpu.VMEM((WIN * WIN, C), jnp.float32),
            pltpu.VMEM((WIN * WIN, C), jnp.float32),
            pltpu.VMEM((NBIN, C), jnp.float32),
            pltpu.VMEM((NBIN, C), jnp.float32),
            pltpu.SemaphoreType.DMA,
            pltpu.SemaphoreType.DMA,
            pltpu.SemaphoreType.DMA,
            pltpu.SemaphoreType.DMA,
        ],
    )(_sc_body)
    return f(meta, feat_rows)


def kernel(features, rois, roi_indices):
    B, C, H, W = features.shape
    N = rois.shape[0]
    rois_i = (rois * SPATIAL_SCALE).astype(jnp.int32)
    img = roi_indices.astype(jnp.int32)
    hx, wy = rois_i[:, 0], rois_i[:, 1]
    lh = jnp.clip(rois_i[:, 2] - hx, 1, WIN)
    lw = jnp.clip(rois_i[:, 3] - wy, 1, WIN)
    ncells = lh * lw
    cls = (ncells + 15) // 16 - 1

    i8 = jnp.arange(WIN, dtype=jnp.int32)
    ii = i8[None, :, None]
    jj = i8[None, None, :]
    pos = ii * lw[:, None, None] + jj
    valid = (ii < lh[:, None, None]) & (jj < lw[:, None, None])
    rowid = (img * (H * W) + hx * W + wy)[:, None, None] + ii * W + jj
    rowid = jnp.clip(rowid, 0, B * H * W - 1)
    pos = jnp.where(valid, pos, WIN * WIN)
    base = jnp.broadcast_to(
        (img * (H * W) + hx * W + wy)[:, None], (N, WIN * WIN + 1)
    )
    nidx = jnp.broadcast_to(jnp.arange(N, dtype=jnp.int32)[:, None, None],
                            (N, WIN, WIN))
    idx_rows = base.at[nidx, pos].set(rowid)[:, : WIN * WIN]

    def bounds(l, n_out):
        i = jnp.arange(n_out, dtype=jnp.int32)[None, :]
        r0 = (i * l[:, None]) // n_out
        r1m = -(((-(i + 1)) * l[:, None]) // n_out) - 1
        r1m = jnp.clip(jnp.maximum(r1m, r0), 0, None)
        return r0, jnp.minimum(r1m, l[:, None] - 1)

    x0, x1 = bounds(lh, OUT_H)
    y0, y1 = bounds(lw, OUT_W)
    pw = lw[:, None, None]
    corners = jnp.stack(
        [
            x0[:, :, None] * pw + y0[:, None, :],
            x0[:, :, None] * pw + y1[:, None, :],
            x1[:, :, None] * pw + y0[:, None, :],
            x1[:, :, None] * pw + y1[:, None, :],
        ],
        axis=3,
    ).reshape(N, NBIN, 4)
    cells = jnp.zeros((N, NBIN + 1, 16), jnp.int32).at[:, :NBIN, :4].set(corners)
    hdr = jnp.zeros((N, HDR), jnp.int32).at[:, 0].set(cls)
    meta = jnp.concatenate(
        [hdr, idx_rows, cells.reshape(N, 16 * (NBIN + 1))], axis=1
    )

    feat_rows = features.transpose(0, 2, 3, 1).reshape(B * H * W, C)
    out = _roi_pool_sc(feat_rows, meta)
    return out.transpose(0, 2, 1).reshape(N, C, OUT_H, OUT_W)


# speedup vs baseline: 1.2585x; 1.2334x over previous
import functools

import jax
import jax.numpy as jnp
from jax import lax
from jax.experimental import pallas as pl
from jax.experimental.pallas import tpu as pltpu
from jax.experimental.pallas import tpu_sc as plsc

OUT_H, OUT_W = 7, 7
NBIN = OUT_H * OUT_W
SPATIAL_SCALE = 1.0 / 16.0
WIN = 8
NW = 32
CORN0 = WIN * WIN
META_W = CORN0 + 16 * (NBIN + 1)


def _sc_body(meta_hbm, feat_hbm, out_hbm, m0, m1, reg0, reg1, o0, o1,
             gs0, gs1, os0, os1):
    C = feat_hbm.shape[1]
    NCH = C // 16
    N = meta_hbm.shape[0]
    ms, regs, outs = [m0, m1], [reg0, reg1], [o0, o1]
    gsems, osems = [gs0, gs1], [os0, os1]
    wid = lax.axis_index("s") * 2 + lax.axis_index("c")
    count = (N - 1 - wid) // NW + 1

    def issue(t, b):
        pltpu.sync_copy(meta_hbm.at[t * NW + wid], ms[b])
        pltpu.async_copy(
            feat_hbm.at[ms[b].at[pl.ds(0, WIN * WIN)]], regs[b], gsems[b]
        )

    def gather_wait(b):
        pltpu.make_async_copy(
            feat_hbm.at[ms[b].at[pl.ds(0, WIN * WIN)]], regs[b], gsems[b]
        ).wait()

    def out_wait(b):
        pltpu.make_async_copy(outs[b], out_hbm.at[0], osems[b]).wait()

    @pl.when(count > 0)
    def _prologue():
        issue(0, 0)

    def pair_body(g, carry):
        for b in range(2):
            t = 2 * g + b
            nxt = 1 - b

            @pl.when(t + 1 < count)
            def _prefetch():
                issue(t + 1, nxt)

            @pl.when(t < count)
            def _process():
                gather_wait(b)

                @pl.when(t >= 2)
                def _drain():
                    out_wait(b)

                meta_v, reg_v, out_v = ms[b], regs[b], outs[b]

                def corners_of(ij):
                    cv = meta_v[pl.ds(CORN0 + ij * 16, 16)]
                    return cv[0], cv[1], cv[2], cv[3]

                def bin_body(ij, carry2):
                    c0, c1, c2, c3 = carry2
                    nxt_c = corners_of(ij + 1)
                    for c in range(NCH):
                        sl = pl.ds(c * 16, 16)
                        v = jnp.maximum(
                            jnp.maximum(reg_v[c0, sl], reg_v[c1, sl]),
                            jnp.maximum(reg_v[c2, sl], reg_v[c3, sl]),
                        )
                        out_v[ij, sl] = v
                    return nxt_c

                lax.fori_loop(0, NBIN, bin_body, corners_of(0))
                pltpu.async_copy(out_v, out_hbm.at[t * NW + wid], osems[b])

        return carry

    lax.fori_loop(0, (count + 1) // 2, pair_body, 0)

    @pl.when(count >= 1)
    def _drain0():
        out_wait(0)

    @pl.when(count >= 2)
    def _drain1():
        out_wait(1)


@jax.jit
def _roi_pool_sc(feat_rows, meta):
    R, C = feat_rows.shape
    N = meta.shape[0]
    mesh = plsc.VectorSubcoreMesh(core_axis_name="c", subcore_axis_name="s")
    f = functools.partial(
        pl.kernel,
        mesh=mesh,
        compiler_params=pltpu.CompilerParams(
            needs_layout_passes=False, use_tc_tiling_on_sc=False
        ),
        out_type=jax.ShapeDtypeStruct((N, NBIN, C), jnp.float32),
        scratch_types=[
            pltpu.VMEM((META_W,), jnp.int32),
            pltpu.VMEM((META_W,), jnp.int32),
            pltpu.VMEM((WIN * WIN, C), jnp.float32),
            pltpu.VMEM((WIN * WIN, C), jnp.float32),
            pltpu.VMEM((NBIN, C), jnp.float32),
            pltpu.VMEM((NBIN, C), jnp.float32),
            pltpu.SemaphoreType.DMA,
            pltpu.SemaphoreType.DMA,
            pltpu.SemaphoreType.DMA,
            pltpu.SemaphoreType.DMA,
        ],
    )(_sc_body)
    return f(meta, feat_rows)


def kernel(features, rois, roi_indices):
    B, C, H, W = features.shape
    N = rois.shape[0]
    rois_i = (rois * SPATIAL_SCALE).astype(jnp.int32)
    img = roi_indices.astype(jnp.int32)
    hx, wy = rois_i[:, 0], rois_i[:, 1]
    lh = rois_i[:, 2] - hx
    lw = rois_i[:, 3] - wy
    hs = jnp.clip(hx, 0, H - WIN)
    ws = jnp.clip(wy, 0, W - WIN)

    p = jnp.arange(WIN * WIN, dtype=jnp.int32)
    idx_rows = (img * (H * W))[:, None] + (hs[:, None] + p[None, :] // WIN) * W \
        + (ws[:, None] + p[None, :] % WIN)

    def bounds(l, start, wstart, n_out):
        i = jnp.arange(n_out, dtype=jnp.int32)[None, :]
        r0 = (i * l[:, None]) // n_out
        r1m = -(((-(i + 1)) * l[:, None]) // n_out) - 1
        r1m = jnp.maximum(r1m, r0)
        off = (start - wstart)[:, None]
        return jnp.clip(r0 + off, 0, WIN - 1), jnp.clip(r1m + off, 0, WIN - 1)

    x0, x1 = bounds(lh, hx, hs, OUT_H)
    y0, y1 = bounds(lw, wy, ws, OUT_W)
    corners = jnp.stack(
        [
            x0[:, :, None, None] * WIN + y0[:, None, :, None],
            x0[:, :, None, None] * WIN + y1[:, None, :, None],
            x1[:, :, None, None] * WIN + y0[:, None, :, None],
            x1[:, :, None, None] * WIN + y1[:, None, :, None],
        ],
        axis=3,
    ).reshape(N, NBIN, 4)
    cells = jnp.zeros((N, NBIN + 1, 16), jnp.int32).at[:, :NBIN, :4].set(corners)
    meta = jnp.concatenate([idx_rows, cells.reshape(N, 16 * (NBIN + 1))], axis=1)

    feat_rows = features.transpose(0, 2, 3, 1).reshape(B * H * W, C)
    out = _roi_pool_sc(feat_rows, meta)
    return out.transpose(0, 2, 1).reshape(N, C, OUT_H, OUT_W)
